# Initial kernel scaffold; baseline (speedup 1.0000x reference)
import jax
import jax.numpy as jnp
from jax.experimental import pallas as pl


def kernel(x, W1, a_src1, a_dst1, b1, W2, a_src2, a_dst2, b2, edge_index):
    raise NotImplementedError("write your pallas kernel here")


# speedup vs baseline: 15.4381x; 15.4381x over previous
import jax
import jax.numpy as jnp
from jax import lax
from jax.experimental import pallas as pl
from jax.experimental.pallas import tpu as pltpu
from jax.experimental.pallas import tpu_sc as plsc

_N = 10000
_IN = 128
_HID = 64
_HEADS = 4
_OUT = 128

_C = 128
_NCH = 162
_BT = _C * _NCH
_EP = _BT * 16
_RPT = 640
_NACC = _RPT * 16
_NT = 10240
_RB = 1024
_GRID = 10


def _prep1_body(x_ref, w1_ref, a1_ref, h_ref, s_ref):
    h = jnp.dot(x_ref[...], w1_ref[...], preferred_element_type=jnp.float32)
    h_ref[...] = h
    s_ref[...] = jnp.dot(h, a1_ref[...], preferred_element_type=jnp.float32)


_prep1 = pl.pallas_call(
    _prep1_body,
    grid=(_GRID,),
    in_specs=[
        pl.BlockSpec((_RB, _IN), lambda i: (i, 0)),
        pl.BlockSpec((_IN, _HEADS * _HID), lambda i: (0, 0)),
        pl.BlockSpec((_HEADS * _HID, 128), lambda i: (0, 0)),
    ],
    out_specs=[
        pl.BlockSpec((_RB, _HEADS * _HID), lambda i: (i, 0)),
        pl.BlockSpec((_RB, 128), lambda i: (i, 0)),
    ],
    out_shape=[
        jax.ShapeDtypeStruct((_N, _HEADS * _HID), jnp.float32),
        jax.ShapeDtypeStruct((_N, 128), jnp.float32),
    ],
)


def _den_sum(d_ref):
    return jnp.sum(d_ref[...], axis=1)


def _mid_body(m0_ref, m1_ref, d0_ref, d1_ref, one_ref, b1_ref, w2_ref,
              a2_ref, h2_ref, s2_ref):
    ds0 = _den_sum(d0_ref)
    ds1 = _den_sum(d1_ref)
    one64 = one_ref[...]
    w2 = w2_ref[...]
    h2 = jnp.zeros((m0_ref.shape[2], 128), jnp.float32)
    for h in range(4):
        m_ref = m0_ref if h < 2 else m1_ref
        ds = ds0 if h < 2 else ds1
        grp = h % 2
        n_h = m_ref[0, grp] + m_ref[1, grp]
        dexp = lax.dot_general(ds[grp:grp + 1], one64, (((0,), (0,)), ((), ())),
                               preferred_element_type=jnp.float32) + 1e-16
        g_h = jnp.maximum(n_h / dexp + b1_ref[h:h + 1, :], 0.0)
        h2 = h2 + jnp.dot(g_h, w2[64 * h:64 * h + 64, :],
                          preferred_element_type=jnp.float32)
    h2_ref[...] = h2
    s2_ref[...] = jnp.dot(h2, a2_ref[...], preferred_element_type=jnp.float32)


_mid = pl.pallas_call(
    _mid_body,
    grid=(_GRID,),
    in_specs=[
        pl.BlockSpec((2, 2, _RB, 64), lambda i: (0, 0, i, 0)),
        pl.BlockSpec((2, 2, _RB, 64), lambda i: (0, 0, i, 0)),
        pl.BlockSpec((2, 16, _RB), lambda i: (0, 0, i)),
        pl.BlockSpec((2, 16, _RB), lambda i: (0, 0, i)),
        pl.BlockSpec((1, 64), lambda i: (0, 0)),
        pl.BlockSpec((4, 64), lambda i: (0, 0)),
        pl.BlockSpec((256, 128), lambda i: (0, 0)),
        pl.BlockSpec((128, 128), lambda i: (0, 0)),
    ],
    out_specs=[
        pl.BlockSpec((_RB, 128), lambda i: (i, 0)),
        pl.BlockSpec((_RB, 128), lambda i: (i, 0)),
    ],
    out_shape=[
        jax.ShapeDtypeStruct((_N, 128), jnp.float32),
        jax.ShapeDtypeStruct((_N, 128), jnp.float32),
    ],
)


def _fin_body(m_ref, d_ref, sa_ref, pa_ref, pb_ref, b2_ref, o_ref):
    nA = m_ref[0, 0] + m_ref[1, 0]
    nB = m_ref[0, 1] + m_ref[1, 1]
    dsum = _den_sum(d_ref)
    dexp = lax.dot_general(dsum, sa_ref[...], (((0,), (0,)), ((), ())),
                           preferred_element_type=jnp.float32) + 1e-16
    nfull = (jnp.dot(nA, pa_ref[...], preferred_element_type=jnp.float32)
             + jnp.dot(nB, pb_ref[...], preferred_element_type=jnp.float32))
    o_ref[...] = nfull / dexp + b2_ref[0:1, :]


_fin = pl.pallas_call(
    _fin_body,
    grid=(_GRID,),
    in_specs=[
        pl.BlockSpec((2, 2, _RB, 64), lambda i: (0, 0, i, 0)),
        pl.BlockSpec((2, 16, _RB), lambda i: (0, 0, i)),
        pl.BlockSpec((2, 128), lambda i: (0, 0)),
        pl.BlockSpec((64, 128), lambda i: (0, 0)),
        pl.BlockSpec((64, 128), lambda i: (0, 0)),
        pl.BlockSpec((1, 128), lambda i: (0, 0)),
    ],
    out_specs=pl.BlockSpec((_RB, 128), lambda i: (i, 0)),
    out_shape=jax.ShapeDtypeStruct((_N, _OUT), jnp.float32),
)


def _sc_body(hcat_hbm, ts_hbm, td_hbm, src_hbm, dst_hbm, zr_hbm, zt_hbm,
             tok_hbm, ncat_hbm, den_hbm,
             sidx, sidx2, didx, didx2, tb_s, tb_d, tb_den, rows, acc, sem):
    c = lax.axis_index("c")
    s = lax.axis_index("s")
    g = s // 8
    t = s % 8
    base = pl.multiple_of(c * (_EP // 2) + t * _BT, _C)
    goff_t = g * _NT
    goff_a = g * _NACC

    pltpu.sync_copy(ts_hbm.at[pl.ds(pl.multiple_of(goff_t, 8), _NT)], tb_s)
    pltpu.sync_copy(td_hbm.at[pl.ds(pl.multiple_of(goff_t, 8), _NT)], tb_d)
    pltpu.sync_copy(zt_hbm, tb_den)
    rz = pl.multiple_of(s * _RPT, 8)
    pltpu.sync_copy(zr_hbm, acc.at[pl.ds(rz, _RPT)])
    pltpu.sync_copy(zr_hbm, acc.at[pl.ds(rz + _NACC, _RPT)])
    plsc.subcore_barrier()

    def chunk(i, carry):
        off = pl.multiple_of(base + i * _C, _C)
        pltpu.sync_copy(src_hbm.at[pl.ds(off, _C)], sidx)
        pltpu.sync_copy(dst_hbm.at[pl.ds(off, _C)], didx)

        def addoff(q, cc):
            qb = pl.multiple_of(q * 16, 16)
            sidx2[pl.ds(qb, 16)] = sidx[pl.ds(qb, 16)] + goff_t
            didx2[pl.ds(qb, 16)] = didx[pl.ds(qb, 16)] + goff_a
            return cc

        lax.fori_loop(0, _C // 16, addoff, 0)
        pltpu.async_copy(hcat_hbm.at[sidx2], rows, sem).wait()

        def group16(q, cc):
            qb = pl.multiple_of(q * 16, 16)
            idxs = sidx[pl.ds(qb, 16)]
            idxd = didx[pl.ds(qb, 16)]
            a = plsc.load_gather(tb_s, [idxs])
            b = plsc.load_gather(tb_d, [idxd])
            e = a + b
            e = jnp.where(e > 0.0, e, 0.2 * e)
            w16 = jnp.exp(e)
            plsc.addupdate_scatter(tb_den, [idxd], w16)
            for k in range(16):
                ws = w16[k]
                for j in range(4):
                    sl = pl.ds(16 * j, 16)
                    rows[qb + k, sl] = rows[qb + k, sl] * ws
            return cc

        lax.fori_loop(0, _C // 16, group16, 0)
        pltpu.sync_copy(rows, acc.at[didx2], add=True)
        return carry

    lax.fori_loop(0, _NCH, chunk, 0)
    plsc.subcore_barrier()
    pltpu.sync_copy(acc.at[pl.ds(rz, _RPT)], ncat_hbm.at[c, pl.ds(rz, _RPT), :])
    pltpu.sync_copy(acc.at[pl.ds(rz + _NACC, _RPT)],
                    ncat_hbm.at[c, pl.ds(rz + _NACC, _RPT), :])
    doff = pl.multiple_of(((g * 2 + c) * 8 + t) * _NT, 8)
    pltpu.sync_copy(tb_den, den_hbm.at[pl.ds(doff, _NT)])


_sc_pass = pl.kernel(
    _sc_body,
    out_type=(
        jax.ShapeDtypeStruct((2, 2 * _NACC, 64), jnp.float32),
        jax.ShapeDtypeStruct((2 * 16 * _NT,), jnp.float32),
    ),
    mesh=plsc.VectorSubcoreMesh(core_axis_name="c", subcore_axis_name="s",
                                num_cores=2, num_subcores=16),
    scratch_types=(
        pltpu.VMEM((_C,), jnp.int32),
        pltpu.VMEM((_C,), jnp.int32),
        pltpu.VMEM((_C,), jnp.int32),
        pltpu.VMEM((_C,), jnp.int32),
        pltpu.VMEM((_NT,), jnp.float32),
        pltpu.VMEM((_NT,), jnp.float32),
        pltpu.VMEM((_NT,), jnp.float32),
        pltpu.VMEM((_C, 64), jnp.float32),
        pltpu.VMEM_SHARED((2 * _NACC, 64), jnp.float32),
        pltpu.SemaphoreType.DMA,
    ),
    compiler_params=pltpu.CompilerParams(needs_layout_passes=False,
                                         use_tc_tiling_on_sc=False),
)


def _pad_tbl(col):
    return jnp.pad(col, (0, _NT - _N))


def _gcat(colA, colB):
    z = jnp.zeros((2 * _NT, 64), jnp.float32)
    return z.at[0:_N].set(colA).at[_NT:_NT + _N].set(colB)


def kernel(x, W1, a_src1, a_dst1, b1, W2, a_src2, a_dst2, b2, edge_index):
    e_real = edge_index.shape[1] + _N
    loop = jnp.arange(_N, dtype=jnp.int32)
    src = jnp.concatenate([
        edge_index[0].astype(jnp.int32), loop,
        jnp.zeros((_EP - e_real,), jnp.int32)])
    dst = jnp.concatenate([
        edge_index[1].astype(jnp.int32), loop,
        jnp.full((_EP - e_real,), _N, jnp.int32)])

    heads_of_col = jnp.arange(_HEADS * _HID, dtype=jnp.int32) // _HID
    onehot_s = (heads_of_col[:, None] == jnp.arange(128)[None, :]).astype(jnp.float32)
    onehot_d = (heads_of_col[:, None] + 16 == jnp.arange(128)[None, :]).astype(jnp.float32)
    A1 = a_src1.reshape(-1, 1) * onehot_s + a_dst1.reshape(-1, 1) * onehot_d
    A2 = jnp.zeros((128, 128), jnp.float32)
    A2 = A2.at[:, 0].set(a_src2.reshape(-1)).at[:, 16].set(a_dst2.reshape(-1))

    colh = jnp.arange(128)[None, :] // 64
    SA = (jnp.arange(2)[:, None] == colh).astype(jnp.float32)
    eye64 = jnp.eye(64, dtype=jnp.float32)
    PA = jnp.concatenate([eye64, jnp.zeros((64, 64), jnp.float32)], axis=1)
    PB = jnp.concatenate([jnp.zeros((64, 64), jnp.float32), eye64], axis=1)
    one64 = jnp.ones((1, 64), jnp.float32)

    zr = jnp.zeros((_RPT, 64), jnp.float32)
    zt = jnp.zeros((_NT,), jnp.float32)

    h1, s1 = _prep1(x, W1, A1)
    m0, d0 = _sc_pass(
        _gcat(h1[:, 0:64], h1[:, 64:128]),
        jnp.concatenate([_pad_tbl(s1[:, 0]), _pad_tbl(s1[:, 1])]),
        jnp.concatenate([_pad_tbl(s1[:, 16]), _pad_tbl(s1[:, 17])]),
        src, dst, zr, zt, jnp.zeros((8,), jnp.float32))
    m1, d1 = _sc_pass(
        _gcat(h1[:, 128:192], h1[:, 192:256]),
        jnp.concatenate([_pad_tbl(s1[:, 2]), _pad_tbl(s1[:, 3])]),
        jnp.concatenate([_pad_tbl(s1[:, 18]), _pad_tbl(s1[:, 19])]),
        src, dst, zr, zt, d0[0:8])

    h2, s2 = _mid(m0.reshape(2, 2, _NACC, 64), m1.reshape(2, 2, _NACC, 64),
                  d0.reshape(2, 16, _NT), d1.reshape(2, 16, _NT),
                  one64, b1.reshape(4, 64), W2, A2)
    ts2 = _pad_tbl(s2[:, 0])
    td2 = _pad_tbl(s2[:, 16])
    m2, d2 = _sc_pass(
        _gcat(h2[:, 0:64], h2[:, 64:128]),
        jnp.concatenate([ts2, ts2]), jnp.concatenate([td2, td2]),
        src, dst, zr, zt, d1[0:8])

    return _fin(m2.reshape(2, 2, _NACC, 64), d2.reshape(2, 16, _NT),
                SA, PA, PB, b2.reshape(1, -1))


# speedup vs baseline: 19.6311x; 1.2716x over previous
import jax
import jax.numpy as jnp
from jax import lax
from jax.experimental import pallas as pl
from jax.experimental.pallas import tpu as pltpu
from jax.experimental.pallas import tpu_sc as plsc

_N = 10000
_IN = 128
_HID = 64
_HEADS = 4
_OUT = 128

_C = 128
_NCH = 162
_BT = _C * _NCH
_EP = _BT * 16
_RPT = 640
_NACC = _RPT * 16
_NT = 10240
_RB = 1024
_GRID = 10


def _prep1_body(x_ref, w1_ref, a1_ref, h_ref, s_ref):
    h = jnp.dot(x_ref[...], w1_ref[...], preferred_element_type=jnp.float32)
    h_ref[...] = h
    s_ref[...] = jnp.dot(h, a1_ref[...], preferred_element_type=jnp.float32)


_prep1 = pl.pallas_call(
    _prep1_body,
    grid=(_GRID,),
    in_specs=[
        pl.BlockSpec((_RB, _IN), lambda i: (i, 0)),
        pl.BlockSpec((_IN, _HEADS * _HID), lambda i: (0, 0)),
        pl.BlockSpec((_HEADS * _HID, 128), lambda i: (0, 0)),
    ],
    out_specs=[
        pl.BlockSpec((_RB, _HEADS * _HID), lambda i: (i, 0)),
        pl.BlockSpec((_RB, 128), lambda i: (i, 0)),
    ],
    out_shape=[
        jax.ShapeDtypeStruct((_N, _HEADS * _HID), jnp.float32),
        jax.ShapeDtypeStruct((_N, 128), jnp.float32),
    ],
)


def _den_sum(d_ref):
    return jnp.sum(d_ref[...], axis=1)


def _mid_body(m0_ref, m1_ref, d0_ref, d1_ref, one_ref, b1_ref, w2_ref,
              a2_ref, h2_ref, s2_ref):
    ds0 = _den_sum(d0_ref)
    ds1 = _den_sum(d1_ref)
    one64 = one_ref[...]
    w2 = w2_ref[...]
    h2 = jnp.zeros((m0_ref.shape[2], 128), jnp.float32)
    for h in range(4):
        m_ref = m0_ref if h < 2 else m1_ref
        ds = ds0 if h < 2 else ds1
        grp = h % 2
        n_h = m_ref[0, grp] + m_ref[1, grp]
        dexp = lax.dot_general(ds[grp:grp + 1], one64, (((0,), (0,)), ((), ())),
                               preferred_element_type=jnp.float32) + 1e-16
        g_h = jnp.maximum(n_h / dexp + b1_ref[h:h + 1, :], 0.0)
        h2 = h2 + jnp.dot(g_h, w2[64 * h:64 * h + 64, :],
                          preferred_element_type=jnp.float32)
    h2_ref[...] = h2
    s2_ref[...] = jnp.dot(h2, a2_ref[...], preferred_element_type=jnp.float32)


_mid = pl.pallas_call(
    _mid_body,
    grid=(_GRID,),
    in_specs=[
        pl.BlockSpec((2, 2, _RB, 64), lambda i: (0, 0, i, 0)),
        pl.BlockSpec((2, 2, _RB, 64), lambda i: (0, 0, i, 0)),
        pl.BlockSpec((2, 16, _RB), lambda i: (0, 0, i)),
        pl.BlockSpec((2, 16, _RB), lambda i: (0, 0, i)),
        pl.BlockSpec((1, 64), lambda i: (0, 0)),
        pl.BlockSpec((4, 64), lambda i: (0, 0)),
        pl.BlockSpec((256, 128), lambda i: (0, 0)),
        pl.BlockSpec((128, 128), lambda i: (0, 0)),
    ],
    out_specs=[
        pl.BlockSpec((_RB, 128), lambda i: (i, 0)),
        pl.BlockSpec((_RB, 128), lambda i: (i, 0)),
    ],
    out_shape=[
        jax.ShapeDtypeStruct((_N, 128), jnp.float32),
        jax.ShapeDtypeStruct((_N, 128), jnp.float32),
    ],
)


def _fin_body(m_ref, d_ref, sa_ref, pa_ref, pb_ref, b2_ref, o_ref):
    nA = m_ref[0, 0] + m_ref[1, 0]
    nB = m_ref[0, 1] + m_ref[1, 1]
    dsum = _den_sum(d_ref)
    dexp = lax.dot_general(dsum, sa_ref[...], (((0,), (0,)), ((), ())),
                           preferred_element_type=jnp.float32) + 1e-16
    nfull = (jnp.dot(nA, pa_ref[...], preferred_element_type=jnp.float32)
             + jnp.dot(nB, pb_ref[...], preferred_element_type=jnp.float32))
    o_ref[...] = nfull / dexp + b2_ref[0:1, :]


_fin = pl.pallas_call(
    _fin_body,
    grid=(_GRID,),
    in_specs=[
        pl.BlockSpec((2, 2, _RB, 64), lambda i: (0, 0, i, 0)),
        pl.BlockSpec((2, 16, _RB), lambda i: (0, 0, i)),
        pl.BlockSpec((2, 128), lambda i: (0, 0)),
        pl.BlockSpec((64, 128), lambda i: (0, 0)),
        pl.BlockSpec((64, 128), lambda i: (0, 0)),
        pl.BlockSpec((1, 128), lambda i: (0, 0)),
    ],
    out_specs=pl.BlockSpec((_RB, 128), lambda i: (i, 0)),
    out_shape=jax.ShapeDtypeStruct((_N, _OUT), jnp.float32),
)


def _sc_body(hcat_hbm, ts_hbm, td_hbm, src_hbm, dst_hbm, zr_hbm, zt_hbm,
             tok_hbm, ncat_hbm, den_hbm,
             sidx0, sidx20, didx0, didx20, rows0,
             sidx1, sidx21, didx1, didx21, rows1,
             tb_s, tb_d, tb_den, acc, sem0, sem1):
    c = lax.axis_index("c")
    s = lax.axis_index("s")
    g = s // 8
    t = s % 8
    base = pl.multiple_of(c * (_EP // 2) + t * _BT, _C)
    goff_t = g * _NT
    goff_a = g * _NACC
    bufs = ((sidx0, sidx20, didx0, didx20, rows0, sem0),
            (sidx1, sidx21, didx1, didx21, rows1, sem1))

    pltpu.sync_copy(ts_hbm.at[pl.ds(pl.multiple_of(goff_t, 8), _NT)], tb_s)
    pltpu.sync_copy(td_hbm.at[pl.ds(pl.multiple_of(goff_t, 8), _NT)], tb_d)
    pltpu.sync_copy(zt_hbm, tb_den)
    rz = pl.multiple_of(s * _RPT, 8)
    pltpu.sync_copy(zr_hbm, acc.at[pl.ds(rz, _RPT)])
    pltpu.sync_copy(zr_hbm, acc.at[pl.ds(rz + _NACC, _RPT)])
    plsc.subcore_barrier()

    def stage(i, b):
        sidx, sidx2, didx, didx2, rows, sem = bufs[b]
        off = pl.multiple_of(base + i * _C, _C)
        pltpu.sync_copy(src_hbm.at[pl.ds(off, _C)], sidx)
        pltpu.sync_copy(dst_hbm.at[pl.ds(off, _C)], didx)

        def addoff(q, cc):
            qb = pl.multiple_of(q * 16, 16)
            sidx2[pl.ds(qb, 16)] = sidx[pl.ds(qb, 16)] + goff_t
            didx2[pl.ds(qb, 16)] = didx[pl.ds(qb, 16)] + goff_a
            return cc

        lax.fori_loop(0, _C // 16, addoff, 0)
        return pltpu.async_copy(hcat_hbm.at[sidx2], rows, sem)

    def work(b):
        sidx, sidx2, didx, didx2, rows, sem = bufs[b]
        pltpu.make_async_copy(hcat_hbm.at[sidx2], rows, sem).wait()

        def group16(q, cc):
            qb = pl.multiple_of(q * 16, 16)
            idxs = sidx[pl.ds(qb, 16)]
            idxd = didx[pl.ds(qb, 16)]
            a = plsc.load_gather(tb_s, [idxs])
            b_ = plsc.load_gather(tb_d, [idxd])
            e = a + b_
            e = jnp.where(e > 0.0, e, 0.2 * e)
            w16 = jnp.exp(e)
            plsc.addupdate_scatter(tb_den, [idxd], w16)
            for k in range(16):
                ws = w16[k]
                for j in range(4):
                    sl = pl.ds(16 * j, 16)
                    rows[qb + k, sl] = rows[qb + k, sl] * ws
            return cc

        lax.fori_loop(0, _C // 16, group16, 0)
        pltpu.sync_copy(rows, acc.at[didx2], add=True)

    stage(0, 0)
    stage(1, 1)

    def pipe(i, carry):
        def even(_):
            work(0)
            stage(i + 2, 0)
            return 0
        def odd(_):
            work(1)
            stage(i + 2, 1)
            return 0
        lax.cond(i % 2 == 0, even, odd, 0)
        return carry

    lax.fori_loop(0, _NCH - 2, pipe, 0)
    work(0)
    work(1)
    plsc.subcore_barrier()
    pltpu.sync_copy(acc.at[pl.ds(rz, _RPT)], ncat_hbm.at[c, pl.ds(rz, _RPT), :])
    pltpu.sync_copy(acc.at[pl.ds(rz + _NACC, _RPT)],
                    ncat_hbm.at[c, pl.ds(rz + _NACC, _RPT), :])
    doff = pl.multiple_of(((g * 2 + c) * 8 + t) * _NT, 8)
    pltpu.sync_copy(tb_den, den_hbm.at[pl.ds(doff, _NT)])


_sc_pass = pl.kernel(
    _sc_body,
    out_type=(
        jax.ShapeDtypeStruct((2, 2 * _NACC, 64), jnp.float32),
        jax.ShapeDtypeStruct((2 * 16 * _NT,), jnp.float32),
    ),
    mesh=plsc.VectorSubcoreMesh(core_axis_name="c", subcore_axis_name="s",
                                num_cores=2, num_subcores=16),
    scratch_types=(
        pltpu.VMEM((_C,), jnp.int32),
        pltpu.VMEM((_C,), jnp.int32),
        pltpu.VMEM((_C,), jnp.int32),
        pltpu.VMEM((_C,), jnp.int32),
        pltpu.VMEM((_C, 64), jnp.float32),
        pltpu.VMEM((_C,), jnp.int32),
        pltpu.VMEM((_C,), jnp.int32),
        pltpu.VMEM((_C,), jnp.int32),
        pltpu.VMEM((_C,), jnp.int32),
        pltpu.VMEM((_C, 64), jnp.float32),
        pltpu.VMEM((_NT,), jnp.float32),
        pltpu.VMEM((_NT,), jnp.float32),
        pltpu.VMEM((_NT,), jnp.float32),
        pltpu.VMEM_SHARED((2 * _NACC, 64), jnp.float32),
        pltpu.SemaphoreType.DMA,
        pltpu.SemaphoreType.DMA,
    ),
    compiler_params=pltpu.CompilerParams(needs_layout_passes=False,
                                         use_tc_tiling_on_sc=False),
)


def _pad_tbl(col):
    return jnp.pad(col, (0, _NT - _N))


def _gcat(colA, colB):
    z = jnp.zeros((2 * _NT, 64), jnp.float32)
    return z.at[0:_N].set(colA).at[_NT:_NT + _N].set(colB)


def kernel(x, W1, a_src1, a_dst1, b1, W2, a_src2, a_dst2, b2, edge_index):
    e_real = edge_index.shape[1] + _N
    loop = jnp.arange(_N, dtype=jnp.int32)
    src = jnp.concatenate([
        edge_index[0].astype(jnp.int32), loop,
        jnp.zeros((_EP - e_real,), jnp.int32)])
    dst = jnp.concatenate([
        edge_index[1].astype(jnp.int32), loop,
        jnp.full((_EP - e_real,), _N, jnp.int32)])

    heads_of_col = jnp.arange(_HEADS * _HID, dtype=jnp.int32) // _HID
    onehot_s = (heads_of_col[:, None] == jnp.arange(128)[None, :]).astype(jnp.float32)
    onehot_d = (heads_of_col[:, None] + 16 == jnp.arange(128)[None, :]).astype(jnp.float32)
    A1 = a_src1.reshape(-1, 1) * onehot_s + a_dst1.reshape(-1, 1) * onehot_d
    A2 = jnp.zeros((128, 128), jnp.float32)
    A2 = A2.at[:, 0].set(a_src2.reshape(-1)).at[:, 16].set(a_dst2.reshape(-1))

    colh = jnp.arange(128)[None, :] // 64
    SA = (jnp.arange(2)[:, None] == colh).astype(jnp.float32)
    eye64 = jnp.eye(64, dtype=jnp.float32)
    PA = jnp.concatenate([eye64, jnp.zeros((64, 64), jnp.float32)], axis=1)
    PB = jnp.concatenate([jnp.zeros((64, 64), jnp.float32), eye64], axis=1)
    one64 = jnp.ones((1, 64), jnp.float32)

    zr = jnp.zeros((_RPT, 64), jnp.float32)
    zt = jnp.zeros((_NT,), jnp.float32)

    h1, s1 = _prep1(x, W1, A1)
    m0, d0 = _sc_pass(
        _gcat(h1[:, 0:64], h1[:, 64:128]),
        jnp.concatenate([_pad_tbl(s1[:, 0]), _pad_tbl(s1[:, 1])]),
        jnp.concatenate([_pad_tbl(s1[:, 16]), _pad_tbl(s1[:, 17])]),
        src, dst, zr, zt, jnp.zeros((8,), jnp.float32))
    m1, d1 = _sc_pass(
        _gcat(h1[:, 128:192], h1[:, 192:256]),
        jnp.concatenate([_pad_tbl(s1[:, 2]), _pad_tbl(s1[:, 3])]),
        jnp.concatenate([_pad_tbl(s1[:, 18]), _pad_tbl(s1[:, 19])]),
        src, dst, zr, zt, d0[0:8])

    h2, s2 = _mid(m0.reshape(2, 2, _NACC, 64), m1.reshape(2, 2, _NACC, 64),
                  d0.reshape(2, 16, _NT), d1.reshape(2, 16, _NT),
                  one64, b1.reshape(4, 64), W2, A2)
    ts2 = _pad_tbl(s2[:, 0])
    td2 = _pad_tbl(s2[:, 16])
    m2, d2 = _sc_pass(
        _gcat(h2[:, 0:64], h2[:, 64:128]),
        jnp.concatenate([ts2, ts2]), jnp.concatenate([td2, td2]),
        src, dst, zr, zt, d1[0:8])

    return _fin(m2.reshape(2, 2, _NACC, 64), d2.reshape(2, 16, _NT),
                SA, PA, PB, b2.reshape(1, -1))
